# baseline (speedup 1.0000x reference)
import functools

import numpy as np
import jax
import jax.numpy as jnp
from jax import lax
from jax.experimental import pallas as pl
from jax.experimental.pallas import tpu as pltpu
from jax.experimental.pallas import tpu_sc as plsc

HIDDEN = 64

_j = np.arange(HIDDEN)
_ang = 2.0 * np.pi * np.outer(_j, _j) / HIDDEN
_WR = np.cos(_ang).astype(np.float32)
_WI = (-np.sin(_ang)).astype(np.float32)


@functools.cache
def _make_sc_gather(B: int, ENT: int, REL: int):
    info = plsc.get_sparse_core_info()
    NC, NS = info.num_cores, info.num_subcores
    NW = NC * NS
    assert B % NW == 0
    bpw = B // NW
    CH = 128
    assert bpw % CH == 0
    nch = bpw // CH
    mesh = plsc.VectorSubcoreMesh(core_axis_name="c", subcore_axis_name="s")

    @functools.partial(
        pl.kernel,
        mesh=mesh,
        compiler_params=pltpu.CompilerParams(use_tc_tiling_on_sc=False),
        out_type=(
            jax.ShapeDtypeStruct((B, HIDDEN), jnp.float32),
            jax.ShapeDtypeStruct((B, HIDDEN), jnp.float32),
            jax.ShapeDtypeStruct((B, HIDDEN), jnp.float32),
        ),
        scratch_types=[
            pltpu.VMEM((bpw,), jnp.int32),
            pltpu.VMEM((bpw,), jnp.int32),
            pltpu.VMEM((bpw,), jnp.int32),
            pltpu.VMEM((bpw, HIDDEN), jnp.float32),
            pltpu.VMEM((bpw, HIDDEN), jnp.float32),
            pltpu.VMEM((bpw, HIDDEN), jnp.float32),
            pltpu.SemaphoreType.DMA,
        ],
    )
    def sc_gather(h_hbm, t_hbm, r_hbm, ent_hbm, rel_hbm,
                  oh, ot, orel, hi_v, ti_v, ri_v, hrow_v, trow_v, rrow_v, sem):
        wid = lax.axis_index("s") * NC + lax.axis_index("c")
        base = wid * bpw
        pltpu.sync_copy(h_hbm.at[pl.ds(base, bpw)], hi_v)
        pltpu.sync_copy(t_hbm.at[pl.ds(base, bpw)], ti_v)
        pltpu.sync_copy(r_hbm.at[pl.ds(base, bpw)], ri_v)
        copies = []
        for c in range(nch):
            sl = pl.ds(c * CH, CH)
            copies.append(pltpu.async_copy(ent_hbm.at[hi_v.at[sl]], hrow_v.at[sl], sem))
            copies.append(pltpu.async_copy(ent_hbm.at[ti_v.at[sl]], trow_v.at[sl], sem))
            copies.append(pltpu.async_copy(rel_hbm.at[ri_v.at[sl]], rrow_v.at[sl], sem))
        for cp in copies:
            cp.wait()
        pltpu.sync_copy(hrow_v, oh.at[pl.ds(base, bpw)])
        pltpu.sync_copy(trow_v, ot.at[pl.ds(base, bpw)])
        pltpu.sync_copy(rrow_v, orel.at[pl.ds(base, bpw)])

    return sc_gather


def _tc_body(h_ref, t_ref, r_ref, wr_ref, wi_ref, out_ref):
    f32 = jnp.float32
    hp = jax.lax.Precision.HIGHEST
    h = h_ref[...]
    t = t_ref[...]
    r = r_ref[...]
    wr = wr_ref[...]
    wi = wi_ref[...]
    hr = jnp.dot(h, wr, precision=hp, preferred_element_type=f32)
    hi = jnp.dot(h, wi, precision=hp, preferred_element_type=f32)
    tr = jnp.dot(t, wr, precision=hp, preferred_element_type=f32)
    ti = jnp.dot(t, wi, precision=hp, preferred_element_type=f32)
    rr = jnp.dot(r, wr, precision=hp, preferred_element_type=f32)
    ri = jnp.dot(r, wi, precision=hp, preferred_element_type=f32)
    p = (hr * tr + hi * ti) * rr + (hr * ti - hi * tr) * ri
    s = jnp.sum(p, axis=1, keepdims=True) * (1.0 / HIDDEN)
    nrm = lax.rsqrt(jnp.maximum(jnp.sum(r * r, axis=1, keepdims=True), 1e-12))
    out_ref[...] = -jax.nn.sigmoid(s * nrm)


def _tc_score(h_e, t_e, r_e, interpret=False):
    B = h_e.shape[0]
    BLK = min(B, 2048)
    assert B % BLK == 0
    wr = jnp.asarray(_WR)
    wi = jnp.asarray(_WI)
    return pl.pallas_call(
        _tc_body,
        grid=(B // BLK,),
        in_specs=[
            pl.BlockSpec((BLK, HIDDEN), lambda i: (i, 0)),
            pl.BlockSpec((BLK, HIDDEN), lambda i: (i, 0)),
            pl.BlockSpec((BLK, HIDDEN), lambda i: (i, 0)),
            pl.BlockSpec((HIDDEN, HIDDEN), lambda i: (0, 0)),
            pl.BlockSpec((HIDDEN, HIDDEN), lambda i: (0, 0)),
        ],
        out_specs=pl.BlockSpec((BLK, 1), lambda i: (i, 0)),
        out_shape=jax.ShapeDtypeStruct((B, 1), jnp.float32),
        interpret=interpret,
    )(h_e, t_e, r_e, wr, wi)


def kernel(h, t, r, ent_embeddings, rel_embeddings):
    h = h.astype(jnp.int32)
    t = t.astype(jnp.int32)
    r = r.astype(jnp.int32)
    B = h.shape[0]
    gather = _make_sc_gather(B, ent_embeddings.shape[0], rel_embeddings.shape[0])
    h_e, t_e, r_e = gather(h, t, r, ent_embeddings, rel_embeddings)
    return _tc_score(h_e, t_e, r_e)
